# Initial kernel scaffold; baseline (speedup 1.0000x reference)
import jax
import jax.numpy as jnp
from jax.experimental import pallas as pl


def kernel(x, fc_weight, bias):
    raise NotImplementedError("write your pallas kernel here")


# speedup vs baseline: 1.2559x; 1.2559x over previous
import functools

import jax
import jax.numpy as jnp
from jax import lax
from jax.experimental import pallas as pl
from jax.experimental.pallas import tpu as pltpu
from jax.experimental.pallas import tpu_sc as plsc

B = 16384
F = 26
FIELD = 100000
NW = 32
BPW = B // NW
E = BPW * F
L = 16
CPW = BPW // L

_mesh = plsc.VectorSubcoreMesh(core_axis_name="c", subcore_axis_name="s")


@functools.partial(
    pl.kernel,
    mesh=_mesh,
    out_type=jax.ShapeDtypeStruct((B,), jnp.float32),
    scratch_types=[
        pltpu.VMEM((F, BPW), jnp.int32),
        pltpu.VMEM((E,), jnp.int32),
        pltpu.VMEM((E,), jnp.float32),
        pltpu.VMEM((BPW,), jnp.float32),
        pltpu.VMEM((L,), jnp.float32),
        pltpu.SemaphoreType.DMA,
    ],
)
def _features_linear_sc(xt_hbm, bias_hbm, tab_hbm, out_hbm,
                        xv, idxv, valv, outv, biasv, sem):
    wid = lax.axis_index("s") * 2 + lax.axis_index("c")
    base = wid * BPW

    pltpu.sync_copy(xt_hbm.at[:, pl.ds(base, BPW)], xv)
    pltpu.sync_copy(bias_hbm, biasv)

    def build(i, _):
        f = i // CPW
        j = (i % CPW) * L
        idxv[pl.ds(i * L, L)] = xv[f, pl.ds(j, L)] + f * FIELD
        return 0

    lax.fori_loop(0, E // L, build, 0)

    pltpu.async_copy(tab_hbm.at[idxv], valv, sem).wait()

    def reduce(c, _):
        j = c * L

        def fstep(f, a):
            return a + valv[pl.ds(f * BPW + j, L)]

        outv[pl.ds(j, L)] = lax.fori_loop(0, F, fstep, biasv[...])
        return 0

    lax.fori_loop(0, CPW, reduce, 0)

    pltpu.sync_copy(outv, out_hbm.at[pl.ds(base, BPW)])


def kernel(x, fc_weight, bias):
    xt = x.astype(jnp.int32).T
    tab = fc_weight.reshape(-1)
    bias16 = jnp.broadcast_to(bias.astype(jnp.float32), (L,))
    out = _features_linear_sc(xt, bias16, tab)
    return out.reshape(B, 1)
